# baseline (speedup 1.0000x reference)
import functools

import jax
import jax.numpy as jnp
from jax import lax
from jax.experimental import pallas as pl
from jax.experimental.pallas import tpu as pltpu
from jax.experimental.pallas import tpu_sc as plsc

BATCH = 16384
D = 16
L = 16


def _build_sc_call():
    info = plsc.get_sparse_core_info()
    nc, ns = info.num_cores, info.num_subcores
    nw = nc * ns
    b_per_w = BATCH // nw
    n_groups = b_per_w // L
    mesh = plsc.VectorSubcoreMesh(core_axis_name="c", subcore_axis_name="s")

    @functools.partial(
        pl.kernel,
        mesh=mesh,
        out_type=(
            jax.ShapeDtypeStruct((BATCH,), jnp.float32),
            jax.ShapeDtypeStruct((BATCH,), jnp.float32),
        ),
        scratch_types=[
            pltpu.VMEM((b_per_w,), jnp.int32),
            pltpu.VMEM((b_per_w,), jnp.int32),
            pltpu.VMEM((b_per_w,), jnp.int32),
            pltpu.VMEM((b_per_w, D), jnp.float32),
            pltpu.VMEM((b_per_w, D), jnp.float32),
            pltpu.VMEM((b_per_w, D), jnp.float32),
            pltpu.VMEM((b_per_w,), jnp.float32),
            pltpu.VMEM((b_per_w,), jnp.float32),
            pltpu.SemaphoreType.DMA,
        ],
        compiler_params=pltpu.CompilerParams(
            needs_layout_passes=False, use_tc_tiling_on_sc=False
        ),
    )
    def sc_kernel(user_hbm, item_i_hbm, item_j_hbm, eu_hbm, ei_hbm,
                  out_i_hbm, out_j_hbm,
                  uidx_v, iidx_v, jidx_v, urows_v, irows_v, jrows_v,
                  oi_v, oj_v, sem):
        wid = lax.axis_index("s") * nc + lax.axis_index("c")
        base = wid * b_per_w

        pltpu.sync_copy(user_hbm.at[pl.ds(base, b_per_w)], uidx_v)
        pltpu.sync_copy(item_i_hbm.at[pl.ds(base, b_per_w)], iidx_v)
        pltpu.sync_copy(item_j_hbm.at[pl.ds(base, b_per_w)], jidx_v)

        cu = pltpu.async_copy(eu_hbm.at[uidx_v], urows_v, sem)
        ci = pltpu.async_copy(ei_hbm.at[iidx_v], irows_v, sem)
        cj = pltpu.async_copy(ei_hbm.at[jidx_v], jrows_v, sem)
        cu.wait()
        ci.wait()
        cj.wait()

        lanes = lax.iota(jnp.int32, L)

        def group(g, carry):
            rows = g * L + lanes
            acc_i = jnp.zeros((L,), jnp.float32)
            acc_j = jnp.zeros((L,), jnp.float32)
            for d in range(D):
                col = jnp.full((L,), d, jnp.int32)
                u_c = plsc.load_gather(urows_v, [rows, col])
                i_c = plsc.load_gather(irows_v, [rows, col])
                j_c = plsc.load_gather(jrows_v, [rows, col])
                acc_i = acc_i + u_c * i_c
                acc_j = acc_j + u_c * j_c
            oi_v[pl.ds(g * L, L)] = acc_i
            oj_v[pl.ds(g * L, L)] = acc_j
            return carry

        lax.fori_loop(0, n_groups, group, 0)

        pltpu.sync_copy(oi_v, out_i_hbm.at[pl.ds(base, b_per_w)])
        pltpu.sync_copy(oj_v, out_j_hbm.at[pl.ds(base, b_per_w)])

    return sc_kernel


_sc_call = _build_sc_call()


@jax.jit
def kernel(user, item_i, item_j, embed_user, embed_item):
    return _sc_call(user, item_i, item_j, embed_user, embed_item)


# speedup vs baseline: 3.2101x; 3.2101x over previous
import functools

import jax
import jax.numpy as jnp
from jax import lax
from jax.experimental import pallas as pl
from jax.experimental.pallas import tpu as pltpu
from jax.experimental.pallas import tpu_sc as plsc

V = 1000000


def _build():
    mesh = plsc.VectorSubcoreMesh(core_axis_name="c", subcore_axis_name="s")
    NC = 2

    @functools.partial(
        pl.kernel, mesh=mesh,
        out_type=(jax.ShapeDtypeStruct((16384,), jnp.float32),
                  jax.ShapeDtypeStruct((16384,), jnp.float32)),
        scratch_types=[pltpu.VMEM((512,), jnp.int32),
                       pltpu.VMEM((512,), jnp.int32),
                       pltpu.VMEM((512,), jnp.int32),
                       [pltpu.VMEM((2, 8, 128), jnp.float32) for _ in range(16)],
                       pltpu.VMEM((8192,), jnp.float32),
                       pltpu.VMEM((8192,), jnp.float32),
                       pltpu.VMEM((8192,), jnp.float32),
                       pltpu.VMEM((512,), jnp.float32),
                       pltpu.VMEM((512,), jnp.float32),
                       pltpu.SemaphoreType.DMA],
        compiler_params=pltpu.CompilerParams(needs_layout_passes=False),
    )
    def k(uh, ih, jh, tabA_hbm, tabB_hbm, oi_hbm, oj_hbm,
          uidx_v, iidx_v, jidx_v, ring, ur_v, ir_v, jr_v, oi_v, oj_v, sem):
        wid = lax.axis_index("s") * NC + lax.axis_index("c")
        base = wid * 512
        pltpu.sync_copy(uh.at[pl.ds(base, 512)], uidx_v)
        pltpu.sync_copy(ih.at[pl.ds(base, 512)], iidx_v)
        pltpu.sync_copy(jh.at[pl.ds(base, 512)], jidx_v)
        lanes = lax.iota(jnp.int32, 16)
        hi = lanes // 8
        lo = jnp.bitwise_and(lanes, 7)

        def gather_pass(idx_ref, tab, rows):
            def fetch(g, carry):
                idxv = idx_ref[pl.ds(g * 16, 16)]
                copies = []
                for k in range(16):
                    r = idxv[k]
                    j = (r // 128) * 128
                    copies.append(pltpu.async_copy(tab.at[:, :, pl.ds(j, 128)], ring[k], sem))
                for c in copies:
                    c.wait()
                for k in range(16):
                    r = idxv[k]
                    c = jnp.bitwise_and(r, 127)
                    cvec = jnp.full((16,), c, jnp.int32)
                    col = plsc.load_gather(ring[k], [hi, lo, cvec])
                    plsc.store_scatter(rows, [lanes * 512 + (g * 16 + k)], col)
                return carry
            lax.fori_loop(0, 32, fetch, 0)

        gather_pass(uidx_v, tabA_hbm, ur_v)
        gather_pass(iidx_v, tabB_hbm, ir_v)
        gather_pass(jidx_v, tabB_hbm, jr_v)

        def group(g, carry):
            acci = jnp.zeros((16,), jnp.float32)
            accj = jnp.zeros((16,), jnp.float32)
            for d in range(16):
                u = ur_v[pl.ds(d * 512 + g * 16, 16)]
                acci = acci + u * ir_v[pl.ds(d * 512 + g * 16, 16)]
                accj = accj + u * jr_v[pl.ds(d * 512 + g * 16, 16)]
            oi_v[pl.ds(g * 16, 16)] = acci
            oj_v[pl.ds(g * 16, 16)] = accj
            return carry

        lax.fori_loop(0, 32, group, 0)
        pltpu.sync_copy(oi_v, oi_hbm.at[pl.ds(base, 512)])
        pltpu.sync_copy(oj_v, oj_hbm.at[pl.ds(base, 512)])

    return k


_sc_call = _build()


@jax.jit
def kernel(user, item_i, item_j, embed_user, embed_item):
    eu3 = embed_user.T.reshape(2, 8, V)
    ei3 = embed_item.T.reshape(2, 8, V)
    return _sc_call(user, item_i, item_j, eu3, ei3)


# speedup vs baseline: 3.7305x; 1.1621x over previous
import functools

import jax
import jax.numpy as jnp
from jax import lax
from jax.experimental import pallas as pl
from jax.experimental.pallas import tpu as pltpu
from jax.experimental.pallas import tpu_sc as plsc

V = 1000000


def _build():
    mesh = plsc.VectorSubcoreMesh(core_axis_name="c", subcore_axis_name="s")
    NC = 2

    @functools.partial(
        pl.kernel, mesh=mesh,
        out_type=(jax.ShapeDtypeStruct((16384,), jnp.float32),
                  jax.ShapeDtypeStruct((16384,), jnp.float32)),
        scratch_types=[pltpu.VMEM((512,), jnp.int32),
                       pltpu.VMEM((512,), jnp.int32),
                       pltpu.VMEM((512,), jnp.int32),
                       [pltpu.VMEM((2, 8, 128), jnp.float32) for _ in range(32)],
                       pltpu.VMEM((8192,), jnp.float32),
                       pltpu.VMEM((8192,), jnp.float32),
                       pltpu.VMEM((8192,), jnp.float32),
                       pltpu.VMEM((512,), jnp.float32),
                       pltpu.VMEM((512,), jnp.float32),
                       pltpu.SemaphoreType.DMA],
        compiler_params=pltpu.CompilerParams(needs_layout_passes=False),
    )
    def k(uh, ih, jh, tabA_hbm, tabB_hbm, oi_hbm, oj_hbm,
          uidx_v, iidx_v, jidx_v, ring, ur_v, ir_v, jr_v, oi_v, oj_v, sem):
        wid = lax.axis_index("s") * NC + lax.axis_index("c")
        base = wid * 512
        pltpu.sync_copy(uh.at[pl.ds(base, 512)], uidx_v)
        pltpu.sync_copy(ih.at[pl.ds(base, 512)], iidx_v)
        pltpu.sync_copy(jh.at[pl.ds(base, 512)], jidx_v)
        lanes = lax.iota(jnp.int32, 16)
        hi = lanes // 8
        lo = jnp.bitwise_and(lanes, 7)

        def gather_pass(idx_ref, tab, rows):
            def fetch(g, carry):
                idxv0 = idx_ref[pl.ds(g * 32, 16)]
                idxv1 = idx_ref[pl.ds(g * 32 + 16, 16)]
                copies = []
                for h, idxv in enumerate((idxv0, idxv1)):
                    for k in range(16):
                        r = idxv[k]
                        j = (r // 128) * 128
                        copies.append(pltpu.async_copy(
                            tab.at[:, :, pl.ds(j, 128)], ring[h * 16 + k], sem))
                for c in copies:
                    c.wait()
                for h, idxv in enumerate((idxv0, idxv1)):
                    for k in range(16):
                        r = idxv[k]
                        c = jnp.bitwise_and(r, 127)
                        cvec = jnp.full((16,), c, jnp.int32)
                        col = plsc.load_gather(ring[h * 16 + k], [hi, lo, cvec])
                        plsc.store_scatter(
                            rows, [lanes * 512 + (g * 32 + h * 16 + k)], col)
                return carry
            lax.fori_loop(0, 16, fetch, 0)

        gather_pass(uidx_v, tabA_hbm, ur_v)
        gather_pass(iidx_v, tabB_hbm, ir_v)
        gather_pass(jidx_v, tabB_hbm, jr_v)

        def group(g, carry):
            acci = jnp.zeros((16,), jnp.float32)
            accj = jnp.zeros((16,), jnp.float32)
            for d in range(16):
                u = ur_v[pl.ds(d * 512 + g * 16, 16)]
                acci = acci + u * ir_v[pl.ds(d * 512 + g * 16, 16)]
                accj = accj + u * jr_v[pl.ds(d * 512 + g * 16, 16)]
            oi_v[pl.ds(g * 16, 16)] = acci
            oj_v[pl.ds(g * 16, 16)] = accj
            return carry

        lax.fori_loop(0, 32, group, 0)
        pltpu.sync_copy(oi_v, oi_hbm.at[pl.ds(base, 512)])
        pltpu.sync_copy(oj_v, oj_hbm.at[pl.ds(base, 512)])

    return k


_sc_call = _build()


@jax.jit
def kernel(user, item_i, item_j, embed_user, embed_item):
    eu3 = embed_user.T.reshape(2, 8, V)
    ei3 = embed_item.T.reshape(2, 8, V)
    return _sc_call(user, item_i, item_j, eu3, ei3)


# speedup vs baseline: 3.8101x; 1.0213x over previous
import functools

import jax
import jax.numpy as jnp
from jax import lax
from jax.experimental import pallas as pl
from jax.experimental.pallas import tpu as pltpu
from jax.experimental.pallas import tpu_sc as plsc

V = 1000000


def _build():
    mesh = plsc.VectorSubcoreMesh(core_axis_name="c", subcore_axis_name="s")
    NC = 2

    @functools.partial(
        pl.kernel, mesh=mesh,
        out_type=(jax.ShapeDtypeStruct((16384,), jnp.float32),
                  jax.ShapeDtypeStruct((16384,), jnp.float32)),
        scratch_types=[pltpu.VMEM((512,), jnp.int32),
                       pltpu.VMEM((512,), jnp.int32),
                       pltpu.VMEM((512,), jnp.int32),
                       [pltpu.VMEM((2, 8, 128), jnp.float32) for _ in range(48)],
                       pltpu.VMEM((8192,), jnp.float32),
                       pltpu.VMEM((8192,), jnp.float32),
                       pltpu.VMEM((8192,), jnp.float32),
                       pltpu.VMEM((512,), jnp.float32),
                       pltpu.VMEM((512,), jnp.float32),
                       pltpu.SemaphoreType.DMA],
        compiler_params=pltpu.CompilerParams(needs_layout_passes=False),
    )
    def k(uh, ih, jh, tabA_hbm, tabB_hbm, oi_hbm, oj_hbm,
          uidx_v, iidx_v, jidx_v, ring, ur_v, ir_v, jr_v, oi_v, oj_v, sem):
        wid = lax.axis_index("s") * NC + lax.axis_index("c")
        base = wid * 512
        pltpu.sync_copy(uh.at[pl.ds(base, 512)], uidx_v)
        pltpu.sync_copy(ih.at[pl.ds(base, 512)], iidx_v)
        pltpu.sync_copy(jh.at[pl.ds(base, 512)], jidx_v)
        lanes = lax.iota(jnp.int32, 16)
        hi = lanes // 8
        lo = jnp.bitwise_and(lanes, 7)

        def gather_pass(idx_ref, tab, rows):
            def fetch(g, carry):
                idxv0 = idx_ref[pl.ds(g * 48, 16)]
                idxv1 = idx_ref[pl.ds(g * 48 + 16, 16)]
                idxv2 = idx_ref[pl.ds(g * 48 + 32, 16)]
                copies = []
                for h, idxv in enumerate((idxv0, idxv1, idxv2)):
                    for k in range(16):
                        r = idxv[k]
                        j = (r // 128) * 128
                        copies.append(pltpu.async_copy(
                            tab.at[:, :, pl.ds(j, 128)], ring[h * 16 + k], sem))
                for c in copies:
                    c.wait()
                for h, idxv in enumerate((idxv0, idxv1, idxv2)):
                    for k in range(16):
                        r = idxv[k]
                        c = jnp.bitwise_and(r, 127)
                        cvec = jnp.full((16,), c, jnp.int32)
                        col = plsc.load_gather(ring[h * 16 + k], [hi, lo, cvec])
                        plsc.store_scatter(
                            rows, [lanes * 512 + (g * 48 + h * 16 + k)], col)
                return carry
            lax.fori_loop(0, 10, fetch, 0)
            def tail(g, carry):
                idxv = idx_ref[pl.ds(480 + g * 16, 16)]
                copies = []
                for k in range(16):
                    r = idxv[k]
                    j = (r // 128) * 128
                    copies.append(pltpu.async_copy(
                        tab.at[:, :, pl.ds(j, 128)], ring[k], sem))
                for c in copies:
                    c.wait()
                for k in range(16):
                    r = idxv[k]
                    c = jnp.bitwise_and(r, 127)
                    cvec = jnp.full((16,), c, jnp.int32)
                    col = plsc.load_gather(ring[k], [hi, lo, cvec])
                    plsc.store_scatter(rows, [lanes * 512 + (480 + g * 16 + k)], col)
                return carry
            lax.fori_loop(0, 2, tail, 0)

        gather_pass(uidx_v, tabA_hbm, ur_v)
        gather_pass(iidx_v, tabB_hbm, ir_v)
        gather_pass(jidx_v, tabB_hbm, jr_v)

        def group(g, carry):
            acci = jnp.zeros((16,), jnp.float32)
            accj = jnp.zeros((16,), jnp.float32)
            for d in range(16):
                u = ur_v[pl.ds(d * 512 + g * 16, 16)]
                acci = acci + u * ir_v[pl.ds(d * 512 + g * 16, 16)]
                accj = accj + u * jr_v[pl.ds(d * 512 + g * 16, 16)]
            oi_v[pl.ds(g * 16, 16)] = acci
            oj_v[pl.ds(g * 16, 16)] = accj
            return carry

        lax.fori_loop(0, 32, group, 0)
        pltpu.sync_copy(oi_v, oi_hbm.at[pl.ds(base, 512)])
        pltpu.sync_copy(oj_v, oj_hbm.at[pl.ds(base, 512)])

    return k


_sc_call = _build()


@jax.jit
def kernel(user, item_i, item_j, embed_user, embed_item):
    eu3 = embed_user.T.reshape(2, 8, V)
    ei3 = embed_item.T.reshape(2, 8, V)
    return _sc_call(user, item_i, item_j, eu3, ei3)


# speedup vs baseline: 3.9414x; 1.0345x over previous
import functools

import jax
import jax.numpy as jnp
from jax import lax
from jax.experimental import pallas as pl
from jax.experimental.pallas import tpu as pltpu
from jax.experimental.pallas import tpu_sc as plsc

V = 1000000


def _build():
    mesh = plsc.VectorSubcoreMesh(core_axis_name="c", subcore_axis_name="s")
    NC = 2

    @functools.partial(
        pl.kernel, mesh=mesh,
        out_type=(jax.ShapeDtypeStruct((16384,), jnp.float32),
                  jax.ShapeDtypeStruct((16384,), jnp.float32)),
        scratch_types=[pltpu.VMEM((512,), jnp.int32),
                       pltpu.VMEM((512,), jnp.int32),
                       pltpu.VMEM((512,), jnp.int32),
                       [pltpu.VMEM((2, 8, 128), jnp.float32) for _ in range(32)],
                       pltpu.VMEM((8192,), jnp.float32),
                       pltpu.VMEM((8192,), jnp.float32),
                       pltpu.VMEM((8192,), jnp.float32),
                       pltpu.VMEM((512,), jnp.float32),
                       pltpu.VMEM((512,), jnp.float32),
                       pltpu.SemaphoreType.DMA,
                       pltpu.SemaphoreType.DMA],
        compiler_params=pltpu.CompilerParams(needs_layout_passes=False),
    )
    def k(uh, ih, jh, tabA_hbm, tabB_hbm, oi_hbm, oj_hbm,
          uidx_v, iidx_v, jidx_v, ring, ur_v, ir_v, jr_v, oi_v, oj_v,
          sem, sem2):
        wid = lax.axis_index("s") * NC + lax.axis_index("c")
        base = wid * 512
        pltpu.sync_copy(uh.at[pl.ds(base, 512)], uidx_v)
        pltpu.sync_copy(ih.at[pl.ds(base, 512)], iidx_v)
        pltpu.sync_copy(jh.at[pl.ds(base, 512)], jidx_v)
        lanes = lax.iota(jnp.int32, 16)
        hi = lanes // 8
        lo = jnp.bitwise_and(lanes, 7)

        def gather_pass(idx_ref, tab, rows):
            def issue_sg(s, slot0, sem_):
                idxv = idx_ref[pl.ds(s * 16, 16)]
                for k in range(16):
                    r = idxv[k]
                    j = (r // 128) * 128
                    pltpu.async_copy(
                        tab.at[:, :, pl.ds(j, 128)], ring[slot0 + k], sem_)

            def extract_sg(s, slot0, sem_):
                for k in range(16):
                    pltpu.make_async_copy(
                        tab.at[:, :, pl.ds(0, 128)], ring[slot0 + k], sem_
                    ).wait()
                idxv = idx_ref[pl.ds(s * 16, 16)]
                for k in range(16):
                    r = idxv[k]
                    c = jnp.bitwise_and(r, 127)
                    cvec = jnp.full((16,), c, jnp.int32)
                    col = plsc.load_gather(ring[slot0 + k], [hi, lo, cvec])
                    plsc.store_scatter(rows, [lanes * 512 + (s * 16 + k)], col)

            issue_sg(0, 0, sem)
            issue_sg(1, 16, sem2)

            def fetch(t, carry):
                s = t * 2
                extract_sg(s, 0, sem)
                issue_sg(s + 2, 0, sem)
                extract_sg(s + 1, 16, sem2)
                issue_sg(s + 3, 16, sem2)
                return carry

            lax.fori_loop(0, 15, fetch, 0)
            extract_sg(30, 0, sem)
            extract_sg(31, 16, sem2)

        gather_pass(uidx_v, tabA_hbm, ur_v)
        gather_pass(iidx_v, tabB_hbm, ir_v)
        gather_pass(jidx_v, tabB_hbm, jr_v)

        def group(g, carry):
            acci = jnp.zeros((16,), jnp.float32)
            accj = jnp.zeros((16,), jnp.float32)
            for d in range(16):
                u = ur_v[pl.ds(d * 512 + g * 16, 16)]
                acci = acci + u * ir_v[pl.ds(d * 512 + g * 16, 16)]
                accj = accj + u * jr_v[pl.ds(d * 512 + g * 16, 16)]
            oi_v[pl.ds(g * 16, 16)] = acci
            oj_v[pl.ds(g * 16, 16)] = accj
            return carry

        lax.fori_loop(0, 32, group, 0)
        pltpu.sync_copy(oi_v, oi_hbm.at[pl.ds(base, 512)])
        pltpu.sync_copy(oj_v, oj_hbm.at[pl.ds(base, 512)])

    return k


_sc_call = _build()


@jax.jit
def kernel(user, item_i, item_j, embed_user, embed_item):
    eu3 = embed_user.T.reshape(2, 8, V)
    ei3 = embed_item.T.reshape(2, 8, V)
    return _sc_call(user, item_i, item_j, eu3, ei3)
